# baseline (speedup 1.0000x reference)
import functools

import jax
import jax.numpy as jnp
from jax import lax
from jax.experimental import pallas as pl
from jax.experimental.pallas import tpu as pltpu
from jax.experimental.pallas import tpu_sc as plsc

_N = 4194304
_NW = 32
_PER_W = _N // _NW
_CHUNKS = 4
_CHUNK = _PER_W // _CHUNKS


def _patch_head(cbuf, buf):
    idxv = lax.broadcasted_iota(jnp.int32, (16,), 0)
    lane0 = idxv == 0
    num = cbuf[pl.ds(0, 16)]
    den = cbuf[pl.ds(16, 16)]
    dscale = jnp.where(lane0, 400000.0, 1.0)
    mscale = jnp.where(lane0, 0.001, 1.0)
    patch = num / (den * dscale) * mscale
    head = buf[pl.ds(0, 16)]
    buf[pl.ds(0, 16)] = jnp.where(idxv < 2, patch, head)


def _sc_body(c_hbm, w_hbm, o_hbm, cbuf, buf0, buf1, isem0, isem1, osem0,
             osem1, csem):
    wid = lax.axis_index("c") * 16 + lax.axis_index("s")
    base = wid * _PER_W
    pltpu.async_copy(c_hbm, cbuf, csem).wait()
    bufs = (buf0, buf1)
    in_sems = (isem0, isem1)
    out_sems = (osem0, osem1)
    in_cps = []
    for j in range(2):
        cp = pltpu.async_copy(
            w_hbm.at[pl.ds(base + j * _CHUNK, _CHUNK)], bufs[j], in_sems[j]
        )
        in_cps.append(cp)
    out_cps = []
    for j in range(_CHUNKS):
        in_cps[j].wait()
        if j == 0:
            @pl.when(wid == 0)
            def _():
                _patch_head(cbuf, bufs[0])
        cp = pltpu.async_copy(
            bufs[j % 2], o_hbm.at[pl.ds(base + j * _CHUNK, _CHUNK)],
            out_sems[j % 2],
        )
        out_cps.append(cp)
        if j + 2 < _CHUNKS:
            out_cps[j].wait()
            cp = pltpu.async_copy(
                w_hbm.at[pl.ds(base + (j + 2) * _CHUNK, _CHUNK)],
                bufs[j % 2],
                in_sems[j % 2],
            )
            in_cps.append(cp)
    out_cps[_CHUNKS - 2].wait()
    out_cps[_CHUNKS - 1].wait()


def kernel(y, w, c, t):
    num = jnp.concatenate([c[9:10], c[11:12], jnp.zeros((14,), jnp.float32)])
    den = jnp.concatenate([c[10:11], c[10:11], jnp.ones((14,), jnp.float32)])
    c32 = jnp.concatenate([num, den])
    k = functools.partial(
        pl.kernel,
        mesh=plsc.VectorSubcoreMesh(core_axis_name="c", subcore_axis_name="s"),
        out_type=jax.ShapeDtypeStruct((_N,), jnp.float32),
        scratch_types=[
            pltpu.VMEM((32,), jnp.float32),
            pltpu.VMEM((_CHUNK,), jnp.float32),
            pltpu.VMEM((_CHUNK,), jnp.float32),
            pltpu.SemaphoreType.DMA,
            pltpu.SemaphoreType.DMA,
            pltpu.SemaphoreType.DMA,
            pltpu.SemaphoreType.DMA,
            pltpu.SemaphoreType.DMA,
        ],
    )(_sc_body)
    return k(c32, w)


# speedup vs baseline: 3.2517x; 3.2517x over previous
import jax
import jax.numpy as jnp
from jax import lax
from jax.experimental import pallas as pl
from jax.experimental.pallas import tpu as pltpu

_N = 4194304
_GRID = 2
_BLOCK = _N // _GRID


def _body(c_ref, w_ref, o_ref):
    o_ref[...] = w_ref[...]

    @pl.when(pl.program_id(0) == 0)
    def _patch():
        a = c_ref[9] / (c_ref[10] * 400000.0) * 0.001
        b = c_ref[11] / c_ref[10]
        head = w_ref[pl.ds(0, 128)]
        idx = lax.broadcasted_iota(jnp.int32, head.shape, 0)
        head = jnp.where(idx == 0, a, head)
        head = jnp.where(idx == 1, b, head)
        o_ref[pl.ds(0, 128)] = head


def kernel(y, w, c, t):
    return pl.pallas_call(
        _body,
        grid=(_GRID,),
        in_specs=[
            pl.BlockSpec(memory_space=pltpu.SMEM),
            pl.BlockSpec((_BLOCK,), lambda i: (i,)),
        ],
        out_specs=pl.BlockSpec((_BLOCK,), lambda i: (i,)),
        out_shape=jax.ShapeDtypeStruct((_N,), jnp.float32),
    )(c, w)
